# baseline (speedup 1.0000x reference)
import jax
import jax.numpy as jnp
from jax.experimental import pallas as pl
from jax.experimental.pallas import tpu as pltpu

_CR = 512
_NC = 2048 // _CR
_ND = 3 * _NC


def _dense_pairs(x_ref, z_ref, xo_ref, zo_ref):
    pairs = []
    for k in range(_NC):
        r = pl.ds(k * _CR, _CR)
        pairs.append((x_ref.at[2, 0, r], xo_ref.at[0, r]))
        pairs.append((x_ref.at[1, 1, r], xo_ref.at[1, r]))
    for k in range(_NC):
        r = pl.ds(k * _CR, _CR)
        pairs.append((z_ref.at[0, 2, r], zo_ref.at[0, r]))
    return pairs


def _body(x_ref, y_ref, z_ref, xo_ref, yo_ref, zo_ref,
          dbuf, ybuf, yob, dsem, ysem, osem, yosem):
    y_in = [
        pltpu.make_async_copy(
            y_ref.at[k, :, pl.ds(0, 128)], ybuf.at[k], ysem.at[k]
        )
        for k in range(4)
    ]
    for c in y_in:
        c.start()

    pairs = _dense_pairs(x_ref, z_ref, xo_ref, zo_ref)
    d_in = [
        pltpu.make_async_copy(src, dbuf.at[i], dsem.at[i])
        for i, (src, _) in enumerate(pairs)
    ]
    for c in d_in:
        c.start()

    d_out = []
    for i, (_, dst) in enumerate(pairs):
        d_in[i].wait()
        c = pltpu.make_async_copy(dbuf.at[i], dst, osem)
        c.start()
        d_out.append(c)

    y_out = []
    for k in range(4):
        y_in[k].wait()
        yob[k, :, 0] = ybuf[k, :, 1]
        yob[k, :, 1] = ybuf[k, :, 0]
        c = pltpu.make_async_copy(yob.at[k], yo_ref.at[k], yosem)
        c.start()
        y_out.append(c)

    for c in d_out:
        c.wait()
    for c in y_out:
        c.wait()


def kernel(x, y, z):
    out_shapes = (
        jax.ShapeDtypeStruct((2, 2048, 1024), jnp.float32),
        jax.ShapeDtypeStruct((4, 4096, 2), jnp.float32),
        jax.ShapeDtypeStruct((1, 2048, 1024), jnp.float32),
    )
    any_spec = pl.BlockSpec(memory_space=pl.ANY)
    x_out, y_out, z_out = pl.pallas_call(
        _body,
        in_specs=[any_spec, any_spec, any_spec],
        out_specs=(any_spec, any_spec, any_spec),
        out_shape=out_shapes,
        scratch_shapes=[
            pltpu.VMEM((_ND, _CR, 1024), jnp.float32),
            pltpu.VMEM((4, 4096, 128), jnp.float32),
            pltpu.VMEM((4, 4096, 2), jnp.float32),
            pltpu.SemaphoreType.DMA((_ND,)),
            pltpu.SemaphoreType.DMA((4,)),
            pltpu.SemaphoreType.DMA,
            pltpu.SemaphoreType.DMA,
        ],
    )(x, y, z)
    return (x_out, y_out, z_out)
